# baseline (speedup 1.0000x reference)
import functools

import jax
import jax.numpy as jnp
from jax import lax
from jax.experimental import pallas as pl
from jax.experimental.pallas import tpu as pltpu
from jax.experimental.pallas import tpu_sc as plsc

_X_MIN = -100.0
_X_MAX = 100.0
_N = 50000
_INV_STEP = 1.0 / ((_X_MAX - _X_MIN) / (_N - 1))

_TAB_PAD = _N + 16
_NC = 2
_NS = 16
_NW = _NC * _NS
_ROWS = 8
_COLS = 2048


def kernel(x, y_table, slope):
    del slope
    d0, d1, d2 = x.shape
    chunks_per_row = d2 // _COLS
    n_chunks_total = (d0 * d1 // _ROWS) * chunks_per_row
    n_chunks = n_chunks_total // _NW
    mesh = plsc.VectorSubcoreMesh(core_axis_name="c", subcore_axis_name="s")

    @functools.partial(
        pl.kernel,
        out_type=jax.ShapeDtypeStruct((d0, d1, d2), jnp.float32),
        mesh=mesh,
        scratch_types=[
            pltpu.VMEM((_TAB_PAD,), jnp.float32),
            pltpu.VMEM((_ROWS, _COLS), jnp.float32),
            pltpu.VMEM((_ROWS, _COLS), jnp.float32),
            pltpu.VMEM((_ROWS, _COLS), jnp.float32),
            pltpu.VMEM((_ROWS, _COLS), jnp.float32),
            pltpu.SemaphoreType.DMA,
            pltpu.SemaphoreType.DMA,
            pltpu.SemaphoreType.DMA,
            pltpu.SemaphoreType.DMA,
        ],
        compiler_params=pltpu.CompilerParams(needs_layout_passes=False),
    )
    def run(x_hbm, tab_hbm, out_hbm, tab_v, in_a, in_b, out_a, out_b,
            isem_a, isem_b, osem_a, osem_b):
        wid = lax.axis_index("s") * _NC + lax.axis_index("c")
        pltpu.sync_copy(tab_hbm, tab_v.at[pl.ds(0, _N)])
        tab_v[pl.ds(_N, 16)] = jnp.zeros((16,), jnp.float32)

        rows_per_dev = d1 // _ROWS

        def src_slice(c):
            g = wid * n_chunks + c
            tile_row = g // chunks_per_row
            col0 = (g % chunks_per_row) * _COLS
            d = tile_row // rows_per_dev
            r0 = (tile_row % rows_per_dev) * _ROWS
            return d, r0, col0

        def start_in(c, ibuf, isem):
            d, r0, col0 = src_slice(c)
            pltpu.async_copy(
                x_hbm.at[d, pl.ds(r0, _ROWS), pl.ds(col0, _COLS)], ibuf, isem)

        def compute(src, dst):
            for r in range(_ROWS):
                @plsc.parallel_loop(0, _COLS, step=128)
                def _(t):
                    for k in range(8):
                        xv = src[r, pl.ds(t + k * 16, 16)]
                        xc = jnp.minimum(jnp.maximum(xv, _X_MIN), _X_MAX)
                        idx_f = (xc - _X_MIN) * _INV_STEP
                        idx = idx_f.astype(jnp.int32)
                        frac = idx_f - idx.astype(jnp.float32)
                        y0 = plsc.load_gather(tab_v, [idx])
                        y1 = plsc.load_gather(tab_v, [idx + 1])
                        approx = y0 + frac * (y1 - y0)
                        dst[r, pl.ds(t + k * 16, 16)] = jnp.where(
                            xv > _X_MAX, xv, approx)

        bufs = ((in_a, out_a, isem_a, osem_a), (in_b, out_b, isem_b, osem_b))

        for b, (ibuf, _, isem, _) in enumerate(bufs):
            start_in(b, ibuf, isem)

        def ring_step(g, carry):
            for b, (ibuf, obuf, isem, osem) in enumerate(bufs):
                c = g * 2 + b
                d, r0, col0 = src_slice(c)
                dst_hbm = out_hbm.at[d, pl.ds(r0, _ROWS), pl.ds(col0, _COLS)]
                pltpu.make_async_copy(
                    x_hbm.at[d, pl.ds(r0, _ROWS), pl.ds(col0, _COLS)],
                    ibuf, isem).wait()

                @pl.when(c >= 2)
                def _():
                    pltpu.make_async_copy(obuf, dst_hbm, osem).wait()

                compute(ibuf, obuf)
                pltpu.async_copy(obuf, dst_hbm, osem)

                @pl.when(c + 2 < n_chunks)
                def _():
                    start_in(c + 2, ibuf, isem)
            return carry

        lax.fori_loop(0, n_chunks // 2, ring_step, 0)
        for b, (_, obuf, _, osem) in enumerate(bufs):
            c = n_chunks - 2 + b
            d, r0, col0 = src_slice(c)
            pltpu.make_async_copy(
                obuf, out_hbm.at[d, pl.ds(r0, _ROWS), pl.ds(col0, _COLS)],
                osem).wait()

    return run(x, y_table)


# speedup vs baseline: 2.2990x; 2.2990x over previous
import functools

import jax
import jax.numpy as jnp
from jax import lax
from jax.experimental import pallas as pl
from jax.experimental.pallas import tpu as pltpu
from jax.experimental.pallas import tpu_sc as plsc

_X_MIN = -100.0
_X_MAX = 100.0
_N = 50000
_INV_STEP = 1.0 / ((_X_MAX - _X_MIN) / (_N - 1))

_TAB_PAD = _N + 16
_NC = 2
_NS = 16
_NW = _NC * _NS
_ROWS = 8
_COLS = 2048


def kernel(x, y_table, slope):
    del slope
    d0, d1, d2 = x.shape
    chunks_per_row = d2 // _COLS
    n_chunks_total = (d0 * d1 // _ROWS) * chunks_per_row
    n_chunks = n_chunks_total // _NW
    mesh = plsc.VectorSubcoreMesh(core_axis_name="c", subcore_axis_name="s")

    @functools.partial(
        pl.kernel,
        out_type=jax.ShapeDtypeStruct((d0, d1, d2), jnp.float32),
        mesh=mesh,
        scratch_types=[
            pltpu.VMEM((_TAB_PAD,), jnp.float32),
            pltpu.VMEM((_ROWS, _COLS), jnp.float32),
            pltpu.VMEM((_ROWS, _COLS), jnp.float32),
            pltpu.VMEM((_ROWS, _COLS), jnp.float32),
            pltpu.VMEM((_ROWS, _COLS), jnp.float32),
            pltpu.SemaphoreType.DMA,
            pltpu.SemaphoreType.DMA,
            pltpu.SemaphoreType.DMA,
            pltpu.SemaphoreType.DMA,
        ],
        compiler_params=pltpu.CompilerParams(needs_layout_passes=False),
    )
    def run(x_hbm, tab_hbm, out_hbm, tab_v, in_a, in_b, out_a, out_b,
            isem_a, isem_b, osem_a, osem_b):
        wid = lax.axis_index("s") * _NC + lax.axis_index("c")
        pltpu.sync_copy(tab_hbm, tab_v.at[pl.ds(0, _N)])
        tab_v[pl.ds(_N, 16)] = jnp.zeros((16,), jnp.float32)

        rows_per_dev = d1 // _ROWS

        def src_slice(c):
            g = wid * n_chunks + c
            tile_row = g // chunks_per_row
            col0 = (g % chunks_per_row) * _COLS
            d = tile_row // rows_per_dev
            r0 = (tile_row % rows_per_dev) * _ROWS
            return d, r0, col0

        def start_in(c, ibuf, isem):
            d, r0, col0 = src_slice(c)
            pltpu.async_copy(
                x_hbm.at[d, pl.ds(r0, _ROWS), pl.ds(col0, _COLS)], ibuf, isem)

        n_tiles = _COLS // 128

        def compute(src, dst):
            @plsc.parallel_loop(0, _ROWS * n_tiles, step=1, unroll=2)
            def _(i):
                r = i // n_tiles
                c0 = (i % n_tiles) * 128
                for k in range(8):
                    xv = src[r, pl.ds(c0 + k * 16, 16)]
                    xc = jnp.minimum(jnp.maximum(xv, _X_MIN), _X_MAX)
                    idx_f = (xc - _X_MIN) * _INV_STEP
                    idx = idx_f.astype(jnp.int32)
                    frac = idx_f - idx.astype(jnp.float32)
                    y0 = plsc.load_gather(tab_v, [idx])
                    y1 = plsc.load_gather(tab_v, [idx + 1])
                    approx = y0 + frac * (y1 - y0)
                    dst[r, pl.ds(c0 + k * 16, 16)] = jnp.where(
                        xv > _X_MAX, xv, approx)

        bufs = ((in_a, out_a, isem_a, osem_a), (in_b, out_b, isem_b, osem_b))

        for b, (ibuf, _, isem, _) in enumerate(bufs):
            start_in(b, ibuf, isem)

        def ring_step(g, carry):
            for b, (ibuf, obuf, isem, osem) in enumerate(bufs):
                c = g * 2 + b
                d, r0, col0 = src_slice(c)
                dst_hbm = out_hbm.at[d, pl.ds(r0, _ROWS), pl.ds(col0, _COLS)]
                pltpu.make_async_copy(
                    x_hbm.at[d, pl.ds(r0, _ROWS), pl.ds(col0, _COLS)],
                    ibuf, isem).wait()

                @pl.when(c >= 2)
                def _():
                    pltpu.make_async_copy(obuf, dst_hbm, osem).wait()

                compute(ibuf, obuf)
                pltpu.async_copy(obuf, dst_hbm, osem)

                @pl.when(c + 2 < n_chunks)
                def _():
                    start_in(c + 2, ibuf, isem)
            return carry

        lax.fori_loop(0, n_chunks // 2, ring_step, 0)
        for b, (_, obuf, _, osem) in enumerate(bufs):
            c = n_chunks - 2 + b
            d, r0, col0 = src_slice(c)
            pltpu.make_async_copy(
                obuf, out_hbm.at[d, pl.ds(r0, _ROWS), pl.ds(col0, _COLS)],
                osem).wait()

    return run(x, y_table)
